# baseline (speedup 1.0000x reference)
import functools

import jax
import jax.numpy as jnp
from jax.experimental import pallas as pl

_D_MODEL = 2048
_N_EXPERTS = 64
_BLKH = 1024


def _route_one(logits, probs_out, idx_out, scores_out):
    m1 = jnp.max(logits, axis=-1, keepdims=True)
    p = jnp.exp(logits - m1)
    probs_out[...] = p / jnp.sum(p, axis=-1, keepdims=True)

    iota = jax.lax.broadcasted_iota(jnp.int32, logits.shape, 1)
    i1 = jnp.min(jnp.where(logits == m1, iota, _N_EXPERTS), axis=-1)
    masked = jnp.where(iota == i1[:, None], -jnp.inf, logits)
    m2 = jnp.max(masked, axis=-1, keepdims=True)
    i2 = jnp.min(jnp.where(masked == m2, iota, _N_EXPERTS), axis=-1)
    idx_out[...] = jnp.concatenate([i1[:, None], i2[:, None]], axis=-1)

    e2 = jnp.exp(m2 - m1)
    den = 1.0 + e2
    scores_out[...] = jnp.concatenate([1.0 / den, e2 / den], axis=-1)


def _router_body(xa_ref, xb_ref, w_ref, probs_ref, idx_ref, scores_ref):
    w = w_ref[...]
    dn = (((1,), (1,)), ((), ()))
    la = jax.lax.dot_general(xa_ref[0], w, dn, preferred_element_type=jnp.float32)
    lb = jax.lax.dot_general(xb_ref[0], w, dn, preferred_element_type=jnp.float32)
    _route_one(la, probs_ref.at[0], idx_ref.at[0], scores_ref.at[0])
    _route_one(lb, probs_ref.at[1], idx_ref.at[1], scores_ref.at[1])


@functools.partial(jax.jit, static_argnames=())
def kernel(x, W_gate):
    b, s, d = x.shape
    tokens = b * s
    half = tokens // 2
    xv = x.reshape(2, half, d)
    grid = (half // _BLKH,)
    probs, idx, scores = pl.pallas_call(
        _router_body,
        grid=grid,
        in_specs=[
            pl.BlockSpec((1, _BLKH, d), lambda i: (0, i, 0)),
            pl.BlockSpec((1, _BLKH, d), lambda i: (1, i, 0)),
            pl.BlockSpec((_N_EXPERTS, d), lambda i: (0, 0)),
        ],
        out_specs=[
            pl.BlockSpec((2, _BLKH, _N_EXPERTS), lambda i: (0, i, 0)),
            pl.BlockSpec((2, _BLKH, 2), lambda i: (0, i, 0)),
            pl.BlockSpec((2, _BLKH, 2), lambda i: (0, i, 0)),
        ],
        out_shape=[
            jax.ShapeDtypeStruct((2, half, _N_EXPERTS), jnp.float32),
            jax.ShapeDtypeStruct((2, half, 2), jnp.int32),
            jax.ShapeDtypeStruct((2, half, 2), jnp.float32),
        ],
    )(xv, xv, W_gate)
    return (
        idx.reshape(b, s, 2),
        scores.reshape(b, s, 2),
        probs.reshape(b, s, _N_EXPERTS),
    )
